# Initial kernel scaffold; baseline (speedup 1.0000x reference)
import jax
import jax.numpy as jnp
from jax.experimental import pallas as pl


def kernel(u_gid, i_gid, edge_src, edge_dst, user_emb, item_emb, W_it, al_it, ar_it, b_it, W_rev, al_rev, ar_rev, b_rev):
    raise NotImplementedError("write your pallas kernel here")


# speedup vs baseline: 25.8760x; 25.8760x over previous
import functools

import jax
import jax.numpy as jnp
from jax import lax
from jax.experimental import pallas as pl
from jax.experimental.pallas import tpu as pltpu
from jax.experimental.pallas import tpu_sc as plsc

NU = 10000
NI = 10000
NN = 10000
E = 160000
HID = 256
H = 8
D = 32
HD = H * D

NC = 2
NS = 16
LANES = 16

ROW = 144
NP = 10240
RPT = NP // NS
FCH = 128
NFCH = RPT // FCH
EPT = E // NS
ECH = 80
NCHUNK = EPT // ECH

_mesh = plsc.VectorSubcoreMesh(core_axis_name="c", subcore_axis_name="s")
_sc_params = pltpu.CompilerParams(use_tc_tiling_on_sc=False)


def _bcast_lane(v, lane):
    idx = jnp.full((LANES, 1), lane, dtype=jnp.int32)
    dn = lax.GatherDimensionNumbers(
        offset_dims=(), collapsed_slice_dims=(0,), start_index_map=(0,))
    return lax.gather(v, idx, dn, (1,),
                      mode=lax.GatherScatterMode.PROMISE_IN_BOUNDS)


GPW = 640
GCH = 128
NGCH = GPW // GCH


@functools.partial(
    pl.kernel, mesh=_mesh, compiler_params=_sc_params,
    out_type=jax.ShapeDtypeStruct((2 * NN + 480, HID), jnp.float32),
    scratch_types=[
        pltpu.VMEM((GCH,), jnp.int32),
        pltpu.VMEM((GCH, HID), jnp.float32),
        pltpu.SemaphoreType.DMA,
    ],
)
def _sc_gather(emb_hbm, idx_hbm, out_hbm, idx_v, rows_v, sem):
    c = lax.axis_index("c")
    s = lax.axis_index("s")
    w = s * NC + c
    for k in range(NGCH):
        base = w * GPW + k * GCH
        pltpu.sync_copy(idx_hbm.at[pl.ds(base, GCH)], idx_v)
        pltpu.async_copy(emb_hbm.at[idx_v], rows_v, sem).wait()
        pltpu.sync_copy(rows_v, out_hbm.at[pl.ds(base, GCH)])


def _tc_transform_body(h0_ref, wb_ref, al_ref, ar_ref,
                       t0_ref, t1_ref, e0_ref, e1_ref):
    y = jnp.dot(h0_ref[...], wb_ref[...], preferred_element_type=jnp.float32)
    hs = y[:, :HD]
    hd = y[:, HD:]
    el = jnp.dot(hs, al_ref[...], preferred_element_type=jnp.float32)
    er = jnp.dot(hd, ar_ref[...], preferred_element_type=jnp.float32)
    nb = hs.shape[0]
    z12 = jnp.zeros((nb, 12), jnp.float32)
    t0_ref[...] = jnp.concatenate([hs[:, :128], el[:, :4], z12], axis=1)
    t1_ref[...] = jnp.concatenate([hs[:, 128:], el[:, 4:], z12], axis=1)
    e0_ref[...] = jnp.concatenate([er[:, :4], z12], axis=1)
    e1_ref[...] = jnp.concatenate([er[:, 4:], z12], axis=1)


def _tc_transform(h0, wb, al, ar):
    B = 1000
    nb = NN // B
    return pl.pallas_call(
        _tc_transform_body,
        grid=(nb,),
        in_specs=[
            pl.BlockSpec((B, HID), lambda i: (i, 0)),
            pl.BlockSpec((HID, 2 * HD), lambda i: (0, 0)),
            pl.BlockSpec((HD, H), lambda i: (0, 0)),
            pl.BlockSpec((HD, H), lambda i: (0, 0)),
        ],
        out_specs=[
            pl.BlockSpec((B, ROW), lambda i: (i, 0)),
            pl.BlockSpec((B, ROW), lambda i: (i, 0)),
            pl.BlockSpec((B, 16), lambda i: (i, 0)),
            pl.BlockSpec((B, 16), lambda i: (i, 0)),
        ],
        out_shape=[
            jax.ShapeDtypeStruct((NN, ROW), jnp.float32),
            jax.ShapeDtypeStruct((NN, ROW), jnp.float32),
            jax.ShapeDtypeStruct((NN, 16), jnp.float32),
            jax.ShapeDtypeStruct((NN, 16), jnp.float32),
        ],
    )(h0, wb, al, ar)


@functools.partial(
    pl.kernel, mesh=_mesh, compiler_params=_sc_params,
    out_type=[
        jax.ShapeDtypeStruct((2 * NP, ROW), jnp.float32),
        jax.ShapeDtypeStruct((2 * NP, ROW), jnp.float32),
    ],
    scratch_types=[
        pltpu.VMEM_SHARED((NP, ROW), jnp.float32),
        pltpu.VMEM((ECH,), jnp.int32),
        pltpu.VMEM((ECH,), jnp.int32),
        pltpu.VMEM((ECH,), jnp.int32),
        pltpu.VMEM((ECH, ROW), jnp.float32),
        pltpu.VMEM((ECH, 16), jnp.float32),
        pltpu.VMEM((FCH, ROW), jnp.float32),
        pltpu.SemaphoreType.DMA,
        pltpu.SemaphoreType.DMA,
    ],
)
def _sc_edges(esrc, edst, acat, bcat, eicat, ercat,
              raw_it, raw_rev,
              acc, sidx, didx, didx2, rows, erb, fbuf,
              sem, sem2):
    c = lax.axis_index("c")
    s = lax.axis_index("s")
    coff = c * NN

    def _zrow(r, carry):
        for j in range(ROW // LANES):
            fbuf[r, pl.ds(j * LANES, LANES)] = jnp.zeros((LANES,), jnp.float32)
        return carry
    lax.fori_loop(0, FCH, _zrow, 0)

    for d in range(2):
        src_hbm = esrc if d == 0 else edst
        dst_hbm = edst if d == 0 else esrc
        stab = acat if d == 0 else bcat
        etab = eicat if d == 0 else ercat
        oraw = raw_it if d == 0 else raw_rev

        r0 = s * RPT
        for k in range(NFCH):
            pltpu.sync_copy(fbuf, acc.at[pl.ds(r0 + k * FCH, FCH)])
        plsc.subcore_barrier()

        def _chunk(k, carry):
            base = s * EPT + k * ECH
            pltpu.sync_copy(src_hbm.at[pl.ds(base, ECH)], sidx)
            pltpu.sync_copy(dst_hbm.at[pl.ds(base, ECH)], didx)
            for v in range(ECH // LANES):
                sl = pl.ds(v * LANES, LANES)
                sidx[sl] = sidx[sl] + coff
                didx2[sl] = didx[sl] + coff
            pltpu.async_copy(stab.at[sidx], rows, sem).wait()
            pltpu.async_copy(etab.at[didx2], erb, sem2).wait()

            def _edge(e, cy):
                er_v = erb[e, :]
                el_v = rows[e, pl.ds(128, LANES)]
                x = el_v + er_v
                x = jnp.maximum(x, x * jnp.float32(0.2))
                w = jnp.exp(x)
                rows[e, pl.ds(128, LANES)] = w
                for h in range(4):
                    wb = _bcast_lane(w, h)
                    s0 = pl.ds(h * 32, LANES)
                    s1 = pl.ds(h * 32 + LANES, LANES)
                    rows[e, s0] = rows[e, s0] * wb
                    rows[e, s1] = rows[e, s1] * wb
                return cy
            lax.fori_loop(0, ECH, _edge, 0)

            pltpu.sync_copy(rows, acc.at[didx], add=True)
            return carry
        lax.fori_loop(0, NCHUNK, _chunk, 0)
        plsc.subcore_barrier()

        for k in range(NFCH):
            rr = r0 + k * FCH
            pltpu.sync_copy(acc.at[pl.ds(rr, FCH)], fbuf)
            pltpu.sync_copy(fbuf, oraw.at[pl.ds(c * NP + rr, FCH)])

        if d == 0:
            lax.fori_loop(0, FCH, _zrow, 0)


def _tc_final_body(rit0_ref, rit1_ref, rrev0_ref, rrev1_ref,
                   bit_ref, brev_ref, s_ref, out_ref, acc_ref):
    p = pl.program_id(0)
    i = pl.program_id(1)

    @pl.when(p == 0)
    def _():
        den8 = jnp.concatenate(
            [rit0_ref[:, 128:132], rit1_ref[:, 128:132]], axis=1)
        rep = jnp.dot(den8, s_ref[...], preferred_element_type=jnp.float32)
        hit = (jnp.concatenate([rit0_ref[:, :128], rit1_ref[:, :128]], axis=1)
               / jnp.maximum(rep, 1e-9))
        colsum = jnp.sum(hit, axis=0, keepdims=True)

        @pl.when(i == 0)
        def _():
            acc_ref[0:1, :] = colsum

        @pl.when(i != 0)
        def _():
            acc_ref[0:1, :] = acc_ref[0:1, :] + colsum

    @pl.when(p == 1)
    def _():
        den8 = jnp.concatenate(
            [rrev0_ref[:, 128:132], rrev1_ref[:, 128:132]], axis=1)
        rep = jnp.dot(den8, s_ref[...], preferred_element_type=jnp.float32)
        left = (jnp.concatenate([rrev0_ref[:, :128], rrev1_ref[:, :128]],
                                axis=1)
                / jnp.maximum(rep, 1e-9)) + brev_ref[...]
        mean = acc_ref[0:1, :] * jnp.float32(1.0 / NN) + bit_ref[...]
        nb = left.shape[0]
        out_ref[...] = jnp.concatenate(
            [left, jnp.broadcast_to(mean, (nb, HD))], axis=1)


def _tc_final(rit0, rit1, rrev0, rrev1, b_it, b_rev, s):
    B = 1000
    nb = NN // B
    blk = pl.BlockSpec((B, ROW), lambda p, i: (i, 0))
    fixed = lambda r, w: pl.BlockSpec((r, w), lambda p, i: (0, 0))
    return pl.pallas_call(
        _tc_final_body,
        grid=(2, nb),
        in_specs=[
            blk, blk, blk, blk,
            fixed(1, HD), fixed(1, HD), fixed(H, HD),
        ],
        out_specs=pl.BlockSpec((B, 2 * HD), lambda p, i: (i, 0)),
        out_shape=jax.ShapeDtypeStruct((NN, 2 * HD), jnp.float32),
        scratch_shapes=[pltpu.VMEM((8, HD), jnp.float32)],
    )(rit0, rit1, rrev0, rrev1, b_it, b_rev, s)


def _attn_mat(a):
    cols = jnp.arange(HD, dtype=jnp.int32)
    mask = (cols[:, None] // D) == jnp.arange(H, dtype=jnp.int32)[None, :]
    return mask.astype(jnp.float32) * a.reshape(HD)[:, None]


def kernel(u_gid, i_gid, edge_src, edge_dst, user_emb, item_emb,
           W_it, al_it, ar_it, b_it, W_rev, al_rev, ar_rev, b_rev):
    u_gid = u_gid.astype(jnp.int32)
    i_gid = i_gid.astype(jnp.int32)
    edge_src = edge_src.astype(jnp.int32)
    edge_dst = edge_dst.astype(jnp.int32)

    emb_cat = jnp.concatenate([user_emb, item_emb], axis=0)
    idx_cat = jnp.concatenate(
        [u_gid, i_gid + NU, jnp.zeros((480,), jnp.int32)])
    h0_cat = _sc_gather(emb_cat, idx_cat)
    h0_u = h0_cat[:NU]
    h0_i = h0_cat[NU:2 * NN]

    al_it_m = _attn_mat(al_it)
    ar_it_m = _attn_mat(ar_it)
    al_rev_m = _attn_mat(al_rev)
    ar_rev_m = _attn_mat(ar_rev)
    wb_u = jnp.concatenate([W_it, W_rev], axis=1)
    wb_i = jnp.concatenate([W_rev, W_it], axis=1)
    a0, a1, er_rev0, er_rev1 = _tc_transform(h0_u, wb_u, al_it_m, ar_rev_m)
    b0, b1, er_it0, er_it1 = _tc_transform(h0_i, wb_i, al_rev_m, ar_it_m)

    acat = jnp.concatenate([a0, a1], axis=0)
    bcat = jnp.concatenate([b0, b1], axis=0)
    eicat = jnp.concatenate([er_it0, er_it1], axis=0)
    ercat = jnp.concatenate([er_rev0, er_rev1], axis=0)

    raw_it, raw_rev = _sc_edges(edge_src, edge_dst, acat, bcat, eicat, ercat)

    smat = (jnp.arange(HD, dtype=jnp.int32)[None, :] // D
            == jnp.arange(H, dtype=jnp.int32)[:, None]).astype(jnp.float32)
    return _tc_final(
        raw_it[:NN], raw_it[NP:NP + NN],
        raw_rev[:NN], raw_rev[NP:NP + NN],
        b_it.reshape(1, HD), b_rev.reshape(1, HD), smat)


# speedup vs baseline: 41.8533x; 1.6175x over previous
import functools

import jax
import jax.numpy as jnp
from jax import lax
from jax.experimental import pallas as pl
from jax.experimental.pallas import tpu as pltpu
from jax.experimental.pallas import tpu_sc as plsc

NU = 10000
NI = 10000
NN = 10000
E = 160000
HID = 256
H = 8
D = 32
HD = H * D

NC = 2
NS = 16
LANES = 16

ROW = 144
NP = 10240
RPT = NP // NS
FCH = 32
NFCH = RPT // FCH
EPT = E // NS
ECH = 80
NSLOT = 2
IBLK = ECH * NSLOT
ITERS = EPT // IBLK

_mesh = plsc.VectorSubcoreMesh(core_axis_name="c", subcore_axis_name="s")
_sc_params = pltpu.CompilerParams(use_tc_tiling_on_sc=False)


def _bcast_lane(v, lane):
    idx = jnp.full((LANES, 1), lane, dtype=jnp.int32)
    dn = lax.GatherDimensionNumbers(
        offset_dims=(), collapsed_slice_dims=(0,), start_index_map=(0,))
    return lax.gather(v, idx, dn, (1,),
                      mode=lax.GatherScatterMode.PROMISE_IN_BOUNDS)


GPW = 640
GCH = 160
NGCH = GPW // GCH


@functools.partial(
    pl.kernel, mesh=_mesh, compiler_params=_sc_params,
    out_type=jax.ShapeDtypeStruct((2 * NN + 480, HID), jnp.float32),
    scratch_types=[
        pltpu.VMEM((GCH,), jnp.int32),
        pltpu.VMEM((GCH,), jnp.int32),
        pltpu.VMEM((GCH, HID), jnp.float32),
        pltpu.VMEM((GCH, HID), jnp.float32),
        pltpu.SemaphoreType.DMA,
        pltpu.SemaphoreType.DMA,
        pltpu.SemaphoreType.DMA,
        pltpu.SemaphoreType.DMA,
    ],
)
def _sc_gather(emb_hbm, idx_hbm, out_hbm, idx0, idx1, rows0, rows1,
               g0, g1, w0, w1):
    c = lax.axis_index("c")
    s = lax.axis_index("s")
    w = s * NC + c
    idxv = (idx0, idx1)
    rows = (rows0, rows1)
    gsem = (g0, g1)
    wsem = (w0, w1)
    gd = [None, None]
    wd = [None, None]
    for k in range(NGCH):
        b = k % 2
        base = w * GPW + k * GCH
        if wd[b] is not None:
            wd[b].wait()
        pltpu.sync_copy(idx_hbm.at[pl.ds(base, GCH)], idxv[b])
        gd[b] = pltpu.async_copy(emb_hbm.at[idxv[b]], rows[b], gsem[b])
        if k >= 1:
            pb = 1 - b
            pbase = w * GPW + (k - 1) * GCH
            gd[pb].wait()
            wd[pb] = pltpu.async_copy(rows[pb], out_hbm.at[pl.ds(pbase, GCH)],
                                      wsem[pb])
    lb = (NGCH - 1) % 2
    gd[lb].wait()
    pltpu.sync_copy(rows[lb], out_hbm.at[pl.ds(w * GPW + (NGCH - 1) * GCH,
                                               GCH)])
    if wd[1 - lb] is not None:
        wd[1 - lb].wait()


def _tc_transform_body(h0_ref, wb_ref, al_ref, ar_ref,
                       t0_ref, t1_ref, e0_ref, e1_ref):
    y = jnp.dot(h0_ref[...], wb_ref[...], preferred_element_type=jnp.float32)
    hs = y[:, :HD]
    hd = y[:, HD:]
    el = jnp.dot(hs, al_ref[...], preferred_element_type=jnp.float32)
    er = jnp.dot(hd, ar_ref[...], preferred_element_type=jnp.float32)
    nb = hs.shape[0]
    z12 = jnp.zeros((nb, 12), jnp.float32)
    t0_ref[...] = jnp.concatenate([hs[:, :128], el[:, :4], z12], axis=1)
    t1_ref[...] = jnp.concatenate([hs[:, 128:], el[:, 4:], z12], axis=1)
    e0_ref[...] = jnp.concatenate([er[:, :4], z12], axis=1)
    e1_ref[...] = jnp.concatenate([er[:, 4:], z12], axis=1)


def _tc_transform(h0, wb, al, ar):
    B = 1000
    nb = NN // B
    return pl.pallas_call(
        _tc_transform_body,
        grid=(nb,),
        in_specs=[
            pl.BlockSpec((B, HID), lambda i: (i, 0)),
            pl.BlockSpec((HID, 2 * HD), lambda i: (0, 0)),
            pl.BlockSpec((HD, H), lambda i: (0, 0)),
            pl.BlockSpec((HD, H), lambda i: (0, 0)),
        ],
        out_specs=[
            pl.BlockSpec((B, ROW), lambda i: (i, 0)),
            pl.BlockSpec((B, ROW), lambda i: (i, 0)),
            pl.BlockSpec((B, 16), lambda i: (i, 0)),
            pl.BlockSpec((B, 16), lambda i: (i, 0)),
        ],
        out_shape=[
            jax.ShapeDtypeStruct((NN, ROW), jnp.float32),
            jax.ShapeDtypeStruct((NN, ROW), jnp.float32),
            jax.ShapeDtypeStruct((NN, 16), jnp.float32),
            jax.ShapeDtypeStruct((NN, 16), jnp.float32),
        ],
    )(h0, wb, al, ar)


@functools.partial(
    pl.kernel, mesh=_mesh, compiler_params=_sc_params,
    out_type=[
        jax.ShapeDtypeStruct((2 * NP, ROW), jnp.float32),
        jax.ShapeDtypeStruct((2 * NP, ROW), jnp.float32),
    ],
    scratch_types=(
        [pltpu.VMEM_SHARED((NP, ROW), jnp.float32)]
        + [pltpu.VMEM((IBLK,), jnp.int32)] * 2
        + [pltpu.VMEM((ECH,), jnp.int32)] * NSLOT
        + [pltpu.VMEM((ECH,), jnp.int32)] * NSLOT
        + [pltpu.VMEM((ECH,), jnp.int32)] * NSLOT
        + [pltpu.VMEM((ECH, ROW), jnp.float32)] * NSLOT
        + [pltpu.VMEM((ECH, 16), jnp.float32)] * NSLOT
        + [pltpu.VMEM((FCH, ROW), jnp.float32)]
        + [pltpu.SemaphoreType.DMA] * NSLOT
        + [pltpu.SemaphoreType.DMA] * NSLOT
        + [pltpu.SemaphoreType.DMA]
    ),
)
def _sc_edges(esrc, edst, acat, bcat, eicat, ercat,
              raw_it, raw_rev,
              acc, sbig, dbig, *rest):
    sidx = rest[0:NSLOT]
    didx = rest[NSLOT:2 * NSLOT]
    didx2 = rest[2 * NSLOT:3 * NSLOT]
    rows = rest[3 * NSLOT:4 * NSLOT]
    erb = rest[4 * NSLOT:5 * NSLOT]
    fbuf = rest[5 * NSLOT]
    gsem = rest[5 * NSLOT + 1:6 * NSLOT + 1]
    ssem = rest[6 * NSLOT + 1:7 * NSLOT + 1]
    isem = rest[7 * NSLOT + 1]

    c = lax.axis_index("c")
    s = lax.axis_index("s")
    coff = c * NN

    def _zrow(r, carry):
        for j in range(ROW // LANES):
            fbuf[r, pl.ds(j * LANES, LANES)] = jnp.zeros((LANES,), jnp.float32)
        return carry
    lax.fori_loop(0, FCH, _zrow, 0)

    def _edges_of(rows_b, erb_b):
        def _edge(e, cy):
            er_v = erb_b[e, :]
            el_v = rows_b[e, pl.ds(128, LANES)]
            x = el_v + er_v
            x = jnp.maximum(x, x * jnp.float32(0.2))
            w = jnp.exp(x)
            rows_b[e, pl.ds(128, LANES)] = w
            for h in range(4):
                wb = _bcast_lane(w, h)
                s0 = pl.ds(h * 32, LANES)
                s1 = pl.ds(h * 32 + LANES, LANES)
                rows_b[e, s0] = rows_b[e, s0] * wb
                rows_b[e, s1] = rows_b[e, s1] * wb
            return cy
        lax.fori_loop(0, ECH, _edge, 0)

    for d in range(2):
        src_hbm = esrc if d == 0 else edst
        dst_hbm = edst if d == 0 else esrc
        stab = acat if d == 0 else bcat
        etab = eicat if d == 0 else ercat
        oraw = raw_it if d == 0 else raw_rev

        r0 = s * RPT
        for k in range(NFCH):
            pltpu.sync_copy(fbuf, acc.at[pl.ds(r0 + k * FCH, FCH)])
        plsc.subcore_barrier()

        pltpu.async_copy(src_hbm.at[pl.ds(s * EPT, IBLK)], sbig, isem)
        pltpu.async_copy(dst_hbm.at[pl.ds(s * EPT, IBLK)], dbig, isem)

        def _iter(j, carry):
            pltpu.make_async_copy(
                src_hbm.at[pl.ds(s * EPT, IBLK)], sbig, isem).wait()
            pltpu.make_async_copy(
                dst_hbm.at[pl.ds(s * EPT, IBLK)], dbig, isem).wait()
            gd = []
            for b in range(NSLOT):
                @pl.when(j > 0)
                def _():
                    pltpu.make_async_copy(
                        rows[b], acc.at[didx[b]], ssem[b]).wait()
                for v in range(ECH // LANES):
                    sl16 = pl.ds(b * ECH + v * LANES, LANES)
                    dsl = pl.ds(v * LANES, LANES)
                    sv = sbig[sl16]
                    dv = dbig[sl16]
                    sidx[b][dsl] = sv + coff
                    didx[b][dsl] = dv
                    didx2[b][dsl] = dv + coff
                g1 = pltpu.async_copy(stab.at[sidx[b]], rows[b], gsem[b])
                g2 = pltpu.async_copy(etab.at[didx2[b]], erb[b], gsem[b])
                gd.append((g1, g2))

            @pl.when(j < ITERS - 1)
            def _():
                nbase = s * EPT + (j + 1) * IBLK
                pltpu.async_copy(src_hbm.at[pl.ds(nbase, IBLK)], sbig, isem)
                pltpu.async_copy(dst_hbm.at[pl.ds(nbase, IBLK)], dbig, isem)

            for b in range(NSLOT):
                gd[b][0].wait()
                gd[b][1].wait()
                _edges_of(rows[b], erb[b])
                pltpu.async_copy(rows[b], acc.at[didx[b]], ssem[b],
                                 add=True)
            return carry
        lax.fori_loop(0, ITERS, _iter, 0)

        tbase = s * EPT + ITERS * IBLK
        pltpu.make_async_copy(rows[0], acc.at[didx[0]], ssem[0]).wait()
        pltpu.sync_copy(src_hbm.at[pl.ds(tbase, ECH)], sidx[0])
        pltpu.sync_copy(dst_hbm.at[pl.ds(tbase, ECH)], didx[0])
        for v in range(ECH // LANES):
            sl = pl.ds(v * LANES, LANES)
            sidx[0][sl] = sidx[0][sl] + coff
            didx2[0][sl] = didx[0][sl] + coff
        pltpu.async_copy(stab.at[sidx[0]], rows[0], gsem[0]).wait()
        pltpu.async_copy(etab.at[didx2[0]], erb[0], gsem[0]).wait()
        _edges_of(rows[0], erb[0])
        pltpu.async_copy(rows[0], acc.at[didx[0]], ssem[0], add=True)

        for b in range(NSLOT):
            pltpu.make_async_copy(rows[b], acc.at[didx[b]], ssem[b]).wait()
        plsc.subcore_barrier()

        for k in range(NFCH):
            rr = r0 + k * FCH
            pltpu.sync_copy(acc.at[pl.ds(rr, FCH)], fbuf)
            pltpu.sync_copy(fbuf, oraw.at[pl.ds(c * NP + rr, FCH)])

        if d == 0:
            lax.fori_loop(0, FCH, _zrow, 0)


def _tc_final_body(rit0_ref, rit1_ref, rrev0_ref, rrev1_ref,
                   bit_ref, brev_ref, s_ref, out_ref, acc_ref):
    p = pl.program_id(0)
    i = pl.program_id(1)

    @pl.when(p == 0)
    def _():
        den8 = jnp.concatenate(
            [rit0_ref[:, 128:132], rit1_ref[:, 128:132]], axis=1)
        rep = jnp.dot(den8, s_ref[...], preferred_element_type=jnp.float32)
        hit = (jnp.concatenate([rit0_ref[:, :128], rit1_ref[:, :128]], axis=1)
               / jnp.maximum(rep, 1e-9))
        colsum = jnp.sum(hit, axis=0, keepdims=True)

        @pl.when(i == 0)
        def _():
            acc_ref[0:1, :] = colsum

        @pl.when(i != 0)
        def _():
            acc_ref[0:1, :] = acc_ref[0:1, :] + colsum

    @pl.when(p == 1)
    def _():
        den8 = jnp.concatenate(
            [rrev0_ref[:, 128:132], rrev1_ref[:, 128:132]], axis=1)
        rep = jnp.dot(den8, s_ref[...], preferred_element_type=jnp.float32)
        left = (jnp.concatenate([rrev0_ref[:, :128], rrev1_ref[:, :128]],
                                axis=1)
                / jnp.maximum(rep, 1e-9)) + brev_ref[...]
        mean = acc_ref[0:1, :] * jnp.float32(1.0 / NN) + bit_ref[...]
        nb = left.shape[0]
        out_ref[...] = jnp.concatenate(
            [left, jnp.broadcast_to(mean, (nb, HD))], axis=1)


def _tc_final(rit0, rit1, rrev0, rrev1, b_it, b_rev, s):
    B = 1000
    nb = NN // B
    blk = pl.BlockSpec((B, ROW), lambda p, i: (i, 0))
    fixed = lambda r, w: pl.BlockSpec((r, w), lambda p, i: (0, 0))
    return pl.pallas_call(
        _tc_final_body,
        grid=(2, nb),
        in_specs=[
            blk, blk, blk, blk,
            fixed(1, HD), fixed(1, HD), fixed(H, HD),
        ],
        out_specs=pl.BlockSpec((B, 2 * HD), lambda p, i: (i, 0)),
        out_shape=jax.ShapeDtypeStruct((NN, 2 * HD), jnp.float32),
        scratch_shapes=[pltpu.VMEM((8, HD), jnp.float32)],
    )(rit0, rit1, rrev0, rrev1, b_it, b_rev, s)


def _attn_mat(a):
    cols = jnp.arange(HD, dtype=jnp.int32)
    mask = (cols[:, None] // D) == jnp.arange(H, dtype=jnp.int32)[None, :]
    return mask.astype(jnp.float32) * a.reshape(HD)[:, None]


def kernel(u_gid, i_gid, edge_src, edge_dst, user_emb, item_emb,
           W_it, al_it, ar_it, b_it, W_rev, al_rev, ar_rev, b_rev):
    u_gid = u_gid.astype(jnp.int32)
    i_gid = i_gid.astype(jnp.int32)
    edge_src = edge_src.astype(jnp.int32)
    edge_dst = edge_dst.astype(jnp.int32)

    emb_cat = jnp.concatenate([user_emb, item_emb], axis=0)
    idx_cat = jnp.concatenate(
        [u_gid, i_gid + NU, jnp.zeros((480,), jnp.int32)])
    h0_cat = _sc_gather(emb_cat, idx_cat)
    h0_u = h0_cat[:NU]
    h0_i = h0_cat[NU:2 * NN]

    al_it_m = _attn_mat(al_it)
    ar_it_m = _attn_mat(ar_it)
    al_rev_m = _attn_mat(al_rev)
    ar_rev_m = _attn_mat(ar_rev)
    wb_u = jnp.concatenate([W_it, W_rev], axis=1)
    wb_i = jnp.concatenate([W_rev, W_it], axis=1)
    a0, a1, er_rev0, er_rev1 = _tc_transform(h0_u, wb_u, al_it_m, ar_rev_m)
    b0, b1, er_it0, er_it1 = _tc_transform(h0_i, wb_i, al_rev_m, ar_it_m)

    acat = jnp.concatenate([a0, a1], axis=0)
    bcat = jnp.concatenate([b0, b1], axis=0)
    eicat = jnp.concatenate([er_it0, er_it1], axis=0)
    ercat = jnp.concatenate([er_rev0, er_rev1], axis=0)

    raw_it, raw_rev = _sc_edges(edge_src, edge_dst, acat, bcat, eicat, ercat)

    smat = (jnp.arange(HD, dtype=jnp.int32)[None, :] // D
            == jnp.arange(H, dtype=jnp.int32)[:, None]).astype(jnp.float32)
    return _tc_final(
        raw_it[:NN], raw_it[NP:NP + NN],
        raw_rev[:NN], raw_rev[NP:NP + NN],
        b_it.reshape(1, HD), b_rev.reshape(1, HD), smat)


# speedup vs baseline: 46.6439x; 1.1145x over previous
import functools

import jax
import jax.numpy as jnp
from jax import lax
from jax.experimental import pallas as pl
from jax.experimental.pallas import tpu as pltpu
from jax.experimental.pallas import tpu_sc as plsc

NU = 10000
NI = 10000
NN = 10000
E = 160000
HID = 256
H = 8
D = 32
HD = H * D

NC = 2
NS = 16
LANES = 16

ROW = 144
NP = 10240
TSTRIDE = 624
FCH = 32
NFCH = 640 // FCH
EPT = E // NS
ECH = 80
NSLOT = 2
IBLK = ECH * NSLOT
ITERS = EPT // IBLK

_mesh = plsc.VectorSubcoreMesh(core_axis_name="c", subcore_axis_name="s")
_sc_params = pltpu.CompilerParams(use_tc_tiling_on_sc=False)


def _bcast_lane(v, lane):
    idx = jnp.full((LANES, 1), lane, dtype=jnp.int32)
    dn = lax.GatherDimensionNumbers(
        offset_dims=(), collapsed_slice_dims=(0,), start_index_map=(0,))
    return lax.gather(v, idx, dn, (1,),
                      mode=lax.GatherScatterMode.PROMISE_IN_BOUNDS)


GSTR = 312
GCHS = (128, 128, 72)


@functools.partial(
    pl.kernel, mesh=_mesh, compiler_params=_sc_params,
    out_type=jax.ShapeDtypeStruct((2 * NN, HID), jnp.float32),
    scratch_types=[
        pltpu.VMEM((128,), jnp.int32),
        pltpu.VMEM((128,), jnp.int32),
        pltpu.VMEM((72,), jnp.int32),
        pltpu.VMEM((72,), jnp.int32),
        pltpu.VMEM((128, HID), jnp.float32),
        pltpu.VMEM((128, HID), jnp.float32),
        pltpu.VMEM((72, HID), jnp.float32),
        pltpu.VMEM((72, HID), jnp.float32),
        pltpu.SemaphoreType.DMA,
        pltpu.SemaphoreType.DMA,
        pltpu.SemaphoreType.DMA,
        pltpu.SemaphoreType.DMA,
    ],
)
def _sc_gather(uemb, iemb, ugid, igid, out_hbm,
               idxf0, idxf1, idxs0, idxs1, rowf0, rowf1, rows0, rows1,
               g0, g1, w0, w1):
    c = lax.axis_index("c")
    s = lax.axis_index("s")
    w = s * NC + c
    wbase = w * GSTR
    idxf = (idxf0, idxf1)
    idxs = (idxs0, idxs1)
    rowf = (rowf0, rowf1)
    rowsm = (rows0, rows1)
    gsem = (g0, g1)
    wsem = (w0, w1)

    jobs = []
    for side in range(2):
        for k, ln in enumerate(GCHS):
            jobs.append((side, side * NN, k, ln))

    gd = [None, None]
    wd = [None, None]
    meta = [None, None]
    for j, (side, ooff, k, ln) in enumerate(jobs):
        b = j % 2
        emb = uemb if side == 0 else iemb
        gid = ugid if side == 0 else igid
        ib = idxf[b] if ln == 128 else idxs[b]
        rb = rowf[b] if ln == 128 else rowsm[b]
        if wd[b] is not None:
            wd[b].wait()
        pltpu.sync_copy(gid.at[pl.ds(wbase + k * 128, ln)], ib)
        gd[b] = pltpu.async_copy(emb.at[ib], rb, gsem[b])
        meta[b] = (rb, ooff + wbase + k * 128, ln)
        if j >= 1:
            pb = 1 - b
            gd[pb].wait()
            prb, obase, oln = meta[pb]
            wd[pb] = pltpu.async_copy(
                prb, out_hbm.at[pl.ds(obase, oln)], wsem[pb])
    lb = (len(jobs) - 1) % 2
    gd[lb].wait()
    prb, obase, oln = meta[lb]
    pltpu.sync_copy(prb, out_hbm.at[pl.ds(obase, oln)])
    if wd[1 - lb] is not None:
        wd[1 - lb].wait()


TB = 1000
TNB = NN // TB


def _tc_transform_body(h0_ref, ws_ref, wd_ref, alf_ref, arf_ref,
                       t_ref, e_ref, yscr):
    p = pl.program_id(1)

    @pl.when(p == 0)
    def _():
        yscr[:, :HD] = jnp.dot(h0_ref[...], ws_ref[...],
                               preferred_element_type=jnp.float32)
        yscr[:, HD:] = jnp.dot(h0_ref[...], wd_ref[...],
                               preferred_element_type=jnp.float32)

    hs = yscr[:, :HD]
    hd = yscr[:, HD:]
    ri = lax.broadcasted_iota(jnp.int32, (HD, H), 0) // D
    ci = lax.broadcasted_iota(jnp.int32, (HD, H), 1)
    mask = (ri == ci).astype(jnp.float32)
    el = jnp.dot(hs * alf_ref[...], mask, preferred_element_type=jnp.float32)
    z12 = jnp.zeros((TB, 12), jnp.float32)

    @pl.when(p == 0)
    def _():
        er = jnp.dot(hd * arf_ref[...], mask,
                     preferred_element_type=jnp.float32)
        e_ref[...] = jnp.concatenate([er, jnp.zeros((TB, 8), jnp.float32)],
                                     axis=1)
        t_ref[...] = jnp.concatenate([hs[:, :128], el[:, :4], z12], axis=1)

    @pl.when(p == 1)
    def _():
        z4 = jnp.zeros((TB, 4), jnp.float32)
        z8 = jnp.zeros((TB, 8), jnp.float32)
        t_ref[...] = jnp.concatenate([hs[:, 128:], z4, el[:, 4:], z8], axis=1)


def _tc_transform(h0_cat, w_src, w_dst, al_flat, ar_flat, side):
    return pl.pallas_call(
        _tc_transform_body,
        grid=(TNB, 2),
        in_specs=[
            pl.BlockSpec((TB, HID), lambda i, p: (side * TNB + i, 0)),
            pl.BlockSpec((HID, HD), lambda i, p: (0, 0)),
            pl.BlockSpec((HID, HD), lambda i, p: (0, 0)),
            pl.BlockSpec((1, HD), lambda i, p: (0, 0)),
            pl.BlockSpec((1, HD), lambda i, p: (0, 0)),
        ],
        out_specs=[
            pl.BlockSpec((TB, ROW), lambda i, p: (p * TNB + i, 0)),
            pl.BlockSpec((TB, 16), lambda i, p: (i, 0)),
        ],
        out_shape=[
            jax.ShapeDtypeStruct((2 * NN, ROW), jnp.float32),
            jax.ShapeDtypeStruct((NN, 16), jnp.float32),
        ],
        scratch_shapes=[pltpu.VMEM((TB, 2 * HD), jnp.float32)],
    )(h0_cat, w_src, w_dst, al_flat, ar_flat)


@functools.partial(
    pl.kernel, mesh=_mesh, compiler_params=_sc_params,
    out_type=[
        jax.ShapeDtypeStruct((2 * NN, ROW), jnp.float32),
        jax.ShapeDtypeStruct((2 * NN, ROW), jnp.float32),
    ],
    scratch_types=(
        [pltpu.VMEM_SHARED((NP, ROW), jnp.float32)]
        + [pltpu.VMEM((IBLK,), jnp.int32)] * 2
        + [pltpu.VMEM((ECH,), jnp.int32)] * NSLOT
        + [pltpu.VMEM((ECH,), jnp.int32)] * NSLOT
        + [pltpu.VMEM((ECH, ROW), jnp.float32)] * NSLOT
        + [pltpu.VMEM((ECH, 16), jnp.float32)] * NSLOT
        + [pltpu.VMEM((FCH, ROW), jnp.float32)]
        + [pltpu.SemaphoreType.DMA] * NSLOT
        + [pltpu.SemaphoreType.DMA] * NSLOT
        + [pltpu.SemaphoreType.DMA]
    ),
)
def _sc_edges(esrc, edst, acat, bcat, erit, errev,
              raw_it, raw_rev,
              acc, sbig, dbig, *rest):
    sidx = rest[0:NSLOT]
    didx = rest[NSLOT:2 * NSLOT]
    rows = rest[2 * NSLOT:3 * NSLOT]
    erb = rest[3 * NSLOT:4 * NSLOT]
    fbuf = rest[4 * NSLOT]
    gsem = rest[4 * NSLOT + 1:5 * NSLOT + 1]
    ssem = rest[5 * NSLOT + 1:6 * NSLOT + 1]
    isem = rest[6 * NSLOT + 1]

    c = lax.axis_index("c")
    s = lax.axis_index("s")
    coff = c * NN
    lane0 = c * 4

    def _zrow(r, carry):
        for j in range(ROW // LANES):
            fbuf[r, pl.ds(j * LANES, LANES)] = jnp.zeros((LANES,), jnp.float32)
        return carry
    lax.fori_loop(0, FCH, _zrow, 0)

    def _edges_of(rows_b, erb_b):
        def _edge(e0, cy):
            for u in range(2):
                e = e0 * 2 + u
                er_v = erb_b[e, :]
                el_v = rows_b[e, pl.ds(128, LANES)]
                x = el_v + er_v
                x = jnp.maximum(x, x * jnp.float32(0.2))
                w = jnp.exp(x)
                rows_b[e, pl.ds(128, LANES)] = w
                for h in range(4):
                    wb = _bcast_lane(w, lane0 + h)
                    s0 = pl.ds(h * 32, LANES)
                    s1 = pl.ds(h * 32 + LANES, LANES)
                    rows_b[e, s0] = rows_b[e, s0] * wb
                    rows_b[e, s1] = rows_b[e, s1] * wb
            return cy
        lax.fori_loop(0, ECH // 2, _edge, 0)

    for d in range(2):
        src_hbm = esrc if d == 0 else edst
        dst_hbm = edst if d == 0 else esrc
        stab = acat if d == 0 else bcat
        etab = erit if d == 0 else errev
        oraw = raw_it if d == 0 else raw_rev

        r0 = s * TSTRIDE
        for k in range(NFCH):
            pltpu.sync_copy(fbuf, acc.at[pl.ds(r0 + k * FCH, FCH)])
        plsc.subcore_barrier()

        pltpu.async_copy(src_hbm.at[pl.ds(s * EPT, IBLK)], sbig, isem)
        pltpu.async_copy(dst_hbm.at[pl.ds(s * EPT, IBLK)], dbig, isem)

        def _iter(j, carry):
            pltpu.make_async_copy(
                src_hbm.at[pl.ds(s * EPT, IBLK)], sbig, isem).wait()
            pltpu.make_async_copy(
                dst_hbm.at[pl.ds(s * EPT, IBLK)], dbig, isem).wait()
            gd = []
            for b in range(NSLOT):
                @pl.when(j > 0)
                def _():
                    pltpu.make_async_copy(
                        rows[b], acc.at[didx[b]], ssem[b]).wait()
                for v in range(ECH // LANES):
                    sl16 = pl.ds(b * ECH + v * LANES, LANES)
                    dsl = pl.ds(v * LANES, LANES)
                    sidx[b][dsl] = sbig[sl16] + coff
                    didx[b][dsl] = dbig[sl16]
                g1 = pltpu.async_copy(stab.at[sidx[b]], rows[b], gsem[b])
                g2 = pltpu.async_copy(etab.at[didx[b]], erb[b], gsem[b])
                gd.append((g1, g2))

            @pl.when(j < ITERS - 1)
            def _():
                nbase = s * EPT + (j + 1) * IBLK
                pltpu.async_copy(src_hbm.at[pl.ds(nbase, IBLK)], sbig, isem)
                pltpu.async_copy(dst_hbm.at[pl.ds(nbase, IBLK)], dbig, isem)

            for b in range(NSLOT):
                gd[b][0].wait()
                gd[b][1].wait()
                _edges_of(rows[b], erb[b])
                pltpu.async_copy(rows[b], acc.at[didx[b]], ssem[b],
                                 add=True)
            return carry
        lax.fori_loop(0, ITERS, _iter, 0)

        tbase = s * EPT + ITERS * IBLK
        pltpu.make_async_copy(rows[0], acc.at[didx[0]], ssem[0]).wait()
        pltpu.sync_copy(src_hbm.at[pl.ds(tbase, ECH)], sidx[0])
        pltpu.sync_copy(dst_hbm.at[pl.ds(tbase, ECH)], didx[0])
        for v in range(ECH // LANES):
            sl = pl.ds(v * LANES, LANES)
            sidx[0][sl] = sidx[0][sl] + coff
        pltpu.async_copy(stab.at[sidx[0]], rows[0], gsem[0]).wait()
        pltpu.async_copy(etab.at[didx[0]], erb[0], gsem[0]).wait()
        _edges_of(rows[0], erb[0])
        pltpu.async_copy(rows[0], acc.at[didx[0]], ssem[0], add=True)

        for b in range(NSLOT):
            pltpu.make_async_copy(rows[b], acc.at[didx[b]], ssem[b]).wait()
        plsc.subcore_barrier()

        for k in range(NFCH):
            rr = r0 + k * FCH
            pltpu.sync_copy(acc.at[pl.ds(rr, FCH)], fbuf)
            pltpu.sync_copy(fbuf, oraw.at[pl.ds(coff + rr, FCH)])

        if d == 0:
            lax.fori_loop(0, FCH, _zrow, 0)


def _tc_final_body(rit0_ref, rit1_ref, rrev0_ref, rrev1_ref,
                   bit_ref, brev_ref, out_ref, acc_ref):
    p = pl.program_id(0)
    i = pl.program_id(1)
    ri = lax.broadcasted_iota(jnp.int32, (H, HD), 0)
    ci = lax.broadcasted_iota(jnp.int32, (H, HD), 1) // D
    smat = (ri == ci).astype(jnp.float32)

    @pl.when(p == 0)
    def _():
        den8 = jnp.concatenate(
            [rit0_ref[:, 128:132], rit1_ref[:, 132:136]], axis=1)
        rep = jnp.dot(den8, smat, preferred_element_type=jnp.float32)
        hit = (jnp.concatenate([rit0_ref[:, :128], rit1_ref[:, :128]], axis=1)
               / jnp.maximum(rep, 1e-9))
        colsum = jnp.sum(hit, axis=0, keepdims=True)

        @pl.when(i == 0)
        def _():
            acc_ref[0:1, :] = colsum

        @pl.when(i != 0)
        def _():
            acc_ref[0:1, :] = acc_ref[0:1, :] + colsum

    @pl.when(p == 1)
    def _():
        den8 = jnp.concatenate(
            [rrev0_ref[:, 128:132], rrev1_ref[:, 132:136]], axis=1)
        rep = jnp.dot(den8, smat, preferred_element_type=jnp.float32)
        left = (jnp.concatenate([rrev0_ref[:, :128], rrev1_ref[:, :128]],
                                axis=1)
                / jnp.maximum(rep, 1e-9)) + brev_ref[...]
        mean = acc_ref[0:1, :] * jnp.float32(1.0 / NN) + bit_ref[...]
        out_ref[...] = jnp.concatenate(
            [left, jnp.broadcast_to(mean, (TB, HD))], axis=1)


def _tc_final(raw_it, raw_rev, b_it, b_rev):
    blk0 = pl.BlockSpec((TB, ROW), lambda p, i: (i, 0))
    blk1 = pl.BlockSpec((TB, ROW), lambda p, i: (i + TNB, 0))
    fixed = pl.BlockSpec((1, HD), lambda p, i: (0, 0))
    return pl.pallas_call(
        _tc_final_body,
        grid=(2, TNB),
        in_specs=[blk0, blk1, blk0, blk1, fixed, fixed],
        out_specs=pl.BlockSpec((TB, 2 * HD), lambda p, i: (i, 0)),
        out_shape=jax.ShapeDtypeStruct((NN, 2 * HD), jnp.float32),
        scratch_shapes=[pltpu.VMEM((8, HD), jnp.float32)],
    )(raw_it, raw_it, raw_rev, raw_rev, b_it, b_rev)


def kernel(u_gid, i_gid, edge_src, edge_dst, user_emb, item_emb,
           W_it, al_it, ar_it, b_it, W_rev, al_rev, ar_rev, b_rev):
    u_gid = u_gid.astype(jnp.int32)
    i_gid = i_gid.astype(jnp.int32)
    edge_src = edge_src.astype(jnp.int32)
    edge_dst = edge_dst.astype(jnp.int32)

    h0_cat = _sc_gather(user_emb, item_emb, u_gid, i_gid)

    acat, er_rev = _tc_transform(h0_cat, W_it, W_rev,
                                 al_it.reshape(1, HD), ar_rev.reshape(1, HD),
                                 side=0)
    bcat, er_it = _tc_transform(h0_cat, W_rev, W_it,
                                al_rev.reshape(1, HD), ar_it.reshape(1, HD),
                                side=1)

    raw_it, raw_rev = _sc_edges(edge_src, edge_dst, acat, bcat, er_it, er_rev)

    return _tc_final(raw_it, raw_rev,
                     b_it.reshape(1, HD), b_rev.reshape(1, HD))
